# baseline (speedup 1.0000x reference)
import functools

import jax
import jax.numpy as jnp
from jax import lax
from jax.experimental import pallas as pl
from jax.experimental.pallas import tpu as pltpu
from jax.experimental.pallas import tpu_sc as plsc

_GAMMA = 12.0
_D = 64
_L = 16
_NPH = 128


@functools.lru_cache(maxsize=None)
def _build_sc_kernel(B, NNEG):
    info = plsc.get_sparse_core_info()
    NC, NS = info.num_cores, info.num_subcores
    NW = NC * NS
    QW = B // NW
    HROWS = B * NNEG // _NPH
    HW = HROWS // NW

    mesh = plsc.VectorSubcoreMesh(core_axis_name="c", subcore_axis_name="s")

    def body(x_hbm, tab_hbm, tgt_hbm, pos_hbm, nidx_hbm,
             plog_hbm, nlog_hbm,
             tgt_v, pos_v, nidx_v, pred_v, posr_v, nbuf_v, plog_v, nlog_v,
             sem_a, sem_n0, sem_n1):
        wid = lax.axis_index("s") * NC + lax.axis_index("c")
        qbase = wid * QW
        hbase = wid * HW
        iota = lax.iota(jnp.int32, _L)

        pltpu.sync_copy(tgt_hbm.at[pl.ds(qbase, QW)], tgt_v)
        pltpu.sync_copy(pos_hbm.at[pl.ds(qbase, QW)], pos_v)
        pltpu.sync_copy(nidx_hbm.at[pl.ds(hbase, HW)], nidx_v)
        pltpu.async_copy(x_hbm.at[tgt_v], pred_v, sem_a).wait()
        pltpu.async_copy(tab_hbm.at[pos_v], posr_v, sem_a).wait()

        def l1_row(ref, j, chunks):
            parts = []
            for c in range(2):
                bits = plsc.bitcast(ref[j, pl.ds(c * 2 * _L, 2 * _L)],
                                    jnp.int32)
                lo = plsc.bitcast(lax.shift_left(bits, 16), jnp.float32)
                hi = plsc.bitcast(
                    jnp.bitwise_and(bits, jnp.int32(-65536)), jnp.float32)
                parts.append(jnp.abs(lo - chunks[2 * c]))
                parts.append(jnp.abs(hi - chunks[2 * c + 1]))
            v = (parts[0] + parts[1]) + (parts[2] + parts[3])
            return jnp.sum(v)

        def pos_group(qg, carry):
            out = jnp.zeros((_L,), jnp.float32)
            for jj in range(_L):
                i = qg * _L + jj
                chunks = [pred_v[i, pl.ds(c * _L, _L)] for c in range(4)]
                s = l1_row(posr_v, i, chunks)
                out = jnp.where(iota == jj, s, out)
            plog_v[pl.ds(qg * _L, _L)] = _GAMMA - out
            return carry
        lax.fori_loop(0, QW // _L, pos_group, 0)

        bufs = [nbuf_v.at[0], nbuf_v.at[1]]
        sems = [sem_n0, sem_n1]

        def start_h(h, par):
            pltpu.make_async_copy(
                tab_hbm.at[nidx_v.at[h]], bufs[par], sems[par]).start()

        def compute_h(h, par):
            q = h // 2
            pltpu.make_async_copy(
                tab_hbm.at[nidx_v.at[h]], bufs[par], sems[par]).wait()
            chunks = [pred_v[q, pl.ds(c * _L, _L)] for c in range(4)]

            def neg_group(g, c2):
                out = jnp.zeros((_L,), jnp.float32)
                for jj in range(_L):
                    s = l1_row(bufs[par], g * _L + jj, chunks)
                    out = jnp.where(iota == jj, s, out)
                nlog_v[h, pl.ds(g * _L, _L)] = _GAMMA - out
                return c2
            lax.fori_loop(0, _NPH // _L, neg_group, 0)

        start_h(0, 0)

        def neg_pair(hh, carry):
            h = hh * 2
            start_h(h + 1, 1)
            compute_h(h, 0)

            @pl.when(h + 2 < HW)
            def _():
                start_h(h + 2, 0)
            compute_h(h + 1, 1)
            return carry
        lax.fori_loop(0, HW // 2, neg_pair, 0)

        pltpu.sync_copy(plog_v, plog_hbm.at[pl.ds(qbase, QW)])
        pltpu.sync_copy(nlog_v, nlog_hbm.at[pl.ds(hbase, HW)])

    return pl.kernel(
        body,
        mesh=mesh,
        compiler_params=pltpu.CompilerParams(
            needs_layout_passes=False, use_tc_tiling_on_sc=False),
        out_type=[
            jax.ShapeDtypeStruct((B,), jnp.float32),
            jax.ShapeDtypeStruct((HROWS, _NPH), jnp.float32),
        ],
        scratch_types=[
            pltpu.VMEM((QW,), jnp.int32),
            pltpu.VMEM((QW,), jnp.int32),
            pltpu.VMEM((HW, _NPH), jnp.int32),
            pltpu.VMEM((QW, _D), jnp.float32),
            pltpu.VMEM((QW, _D), jnp.bfloat16),
            pltpu.VMEM((2, _NPH, _D), jnp.bfloat16),
            pltpu.VMEM((QW,), jnp.float32),
            pltpu.VMEM((HW, _NPH), jnp.float32),
            pltpu.SemaphoreType.DMA,
            pltpu.SemaphoreType.DMA,
            pltpu.SemaphoreType.DMA,
        ],
    )


def kernel(x, entity_table, target_node_idxes, positive_samples, negative_samples):
    B, NNEG = negative_samples.shape
    tgt = target_node_idxes.astype(jnp.int32)
    pos = positive_samples.astype(jnp.int32)
    nidx = negative_samples.astype(jnp.int32).reshape(B * NNEG // _NPH, _NPH)
    D = x.shape[1]
    perm = jnp.concatenate([
        jnp.concatenate([jnp.arange(c * 32, (c + 1) * 32, 2, dtype=jnp.int32),
                         jnp.arange(c * 32 + 1, (c + 1) * 32, 2,
                                    dtype=jnp.int32)])
        for c in range(D // 32)])
    x_p = jnp.take(x, perm, axis=1)
    tab16 = entity_table.astype(jnp.bfloat16)
    sc = _build_sc_kernel(B, NNEG)
    plog, nlog = sc(x_p, tab16, tgt, pos, nidx)
    return plog.reshape(B, 1), nlog.reshape(B, NNEG)


# speedup vs baseline: 1.3572x; 1.3572x over previous
import functools

import jax
import jax.numpy as jnp
from jax import lax
from jax.experimental import pallas as pl
from jax.experimental.pallas import tpu as pltpu
from jax.experimental.pallas import tpu_sc as plsc

_GAMMA = 12.0
_D = 64
_L = 16
_NPH = 128


@functools.lru_cache(maxsize=None)
def _build_sc_kernel(B, NNEG):
    info = plsc.get_sparse_core_info()
    NC, NS = info.num_cores, info.num_subcores
    NW = NC * NS
    QW = B // NW
    HROWS = B * NNEG // _NPH
    HW = HROWS // NW

    mesh = plsc.VectorSubcoreMesh(core_axis_name="c", subcore_axis_name="s")

    def body(x_hbm, tab_hbm, tgt_hbm, pos_hbm, nidx_hbm,
             plog_hbm, nlog_hbm,
             tgt_v, pos_v, nidx_v, pred_v, posr_v, nbuf_v, plog_v, nlog_v,
             sem_a, sem_n0, sem_n1, sem_n2, sem_n3):
        wid = lax.axis_index("s") * NC + lax.axis_index("c")
        qbase = wid * QW
        hbase = wid * HW
        iota = lax.iota(jnp.int32, _L)

        pltpu.sync_copy(tgt_hbm.at[pl.ds(qbase, QW)], tgt_v)
        pltpu.sync_copy(pos_hbm.at[pl.ds(qbase, QW)], pos_v)
        pltpu.sync_copy(nidx_hbm.at[pl.ds(hbase, HW)], nidx_v)
        pltpu.async_copy(x_hbm.at[tgt_v], pred_v, sem_a).wait()
        pltpu.async_copy(tab_hbm.at[pos_v], posr_v, sem_a).wait()

        def l1_row(ref, j, chunks):
            parts = [jnp.abs(ref[j, pl.ds(c * _L, _L)] - chunks[c])
                     for c in range(4)]
            v = (parts[0] + parts[1]) + (parts[2] + parts[3])
            return jnp.sum(v)

        bufs = [nbuf_v.at[k] for k in range(4)]
        sems = [sem_n0, sem_n1, sem_n2, sem_n3]

        def start_h(h, par):
            pltpu.make_async_copy(
                tab_hbm.at[nidx_v.at[h]], bufs[par], sems[par]).start()

        for k in range(3):
            start_h(k, k)

        def pos_group(qg, carry):
            out = jnp.zeros((_L,), jnp.float32)
            for jj in range(_L):
                i = qg * _L + jj
                chunks = [pred_v[i, pl.ds(c * _L, _L)] for c in range(4)]
                s = l1_row(posr_v, i, chunks)
                out = jnp.where(iota == jj, s, out)
            plog_v[pl.ds(qg * _L, _L)] = _GAMMA - out
            return carry
        lax.fori_loop(0, QW // _L, pos_group, 0)

        def compute_h(h, par):
            q = h // 2
            pltpu.make_async_copy(
                tab_hbm.at[nidx_v.at[h]], bufs[par], sems[par]).wait()
            chunks = [pred_v[q, pl.ds(c * _L, _L)] for c in range(4)]

            def neg_group(g, c2):
                out = jnp.zeros((_L,), jnp.float32)
                for jj in range(_L):
                    s = l1_row(bufs[par], g * _L + jj, chunks)
                    out = jnp.where(iota == jj, s, out)
                nlog_v[h, pl.ds(g * _L, _L)] = _GAMMA - out
                return c2
            lax.fori_loop(0, _NPH // _L, neg_group, 0)

        def neg_quad(hh, carry):
            h = hh * 4
            for k in range(4):
                @pl.when(h + k + 3 < HW)
                def _():
                    start_h(h + k + 3, (k + 3) % 4)
                compute_h(h + k, k)
            return carry
        lax.fori_loop(0, HW // 4, neg_quad, 0)

        pltpu.sync_copy(plog_v, plog_hbm.at[pl.ds(qbase, QW)])
        pltpu.sync_copy(nlog_v, nlog_hbm.at[pl.ds(hbase, HW)])

    return pl.kernel(
        body,
        mesh=mesh,
        compiler_params=pltpu.CompilerParams(
            needs_layout_passes=False, use_tc_tiling_on_sc=False),
        out_type=[
            jax.ShapeDtypeStruct((B,), jnp.float32),
            jax.ShapeDtypeStruct((HROWS, _NPH), jnp.float32),
        ],
        scratch_types=[
            pltpu.VMEM((QW,), jnp.int32),
            pltpu.VMEM((QW,), jnp.int32),
            pltpu.VMEM((HW, _NPH), jnp.int32),
            pltpu.VMEM((QW, _D), jnp.float32),
            pltpu.VMEM((QW, _D), jnp.float32),
            pltpu.VMEM((4, _NPH, _D), jnp.float32),
            pltpu.VMEM((QW,), jnp.float32),
            pltpu.VMEM((HW, _NPH), jnp.float32),
            pltpu.SemaphoreType.DMA,
            pltpu.SemaphoreType.DMA,
            pltpu.SemaphoreType.DMA,
            pltpu.SemaphoreType.DMA,
            pltpu.SemaphoreType.DMA,
        ],
    )


def kernel(x, entity_table, target_node_idxes, positive_samples, negative_samples):
    B, NNEG = negative_samples.shape
    tgt = target_node_idxes.astype(jnp.int32)
    pos = positive_samples.astype(jnp.int32)
    nidx = negative_samples.astype(jnp.int32).reshape(B * NNEG // _NPH, _NPH)
    sc = _build_sc_kernel(B, NNEG)
    plog, nlog = sc(x, entity_table, tgt, pos, nidx)
    return plog.reshape(B, 1), nlog.reshape(B, NNEG)
